# baseline (speedup 1.0000x reference)
import jax
import jax.numpy as jnp
from jax.experimental import pallas as pl
from jax.experimental.pallas import tpu as pltpu

_BLOCK = 4096


def _copy_block(pe_ref, out_ref):
    out_ref[...] = pe_ref[...]


def kernel(x, pe):
    seq_len = x.shape[1]
    d = pe.shape[1]
    out = pl.pallas_call(
        _copy_block,
        grid=(seq_len // _BLOCK,),
        in_specs=[pl.BlockSpec((_BLOCK, d), lambda i: (i, 0))],
        out_specs=pl.BlockSpec((_BLOCK, d), lambda i: (i, 0)),
        out_shape=jax.ShapeDtypeStruct((seq_len, d), pe.dtype),
        compiler_params=pltpu.CompilerParams(
            dimension_semantics=("arbitrary",),
        ),
    )(pe)
    return out[None]
